# baseline (speedup 1.0000x reference)
import functools

import numpy as np
import jax
import jax.numpy as jnp
from jax import lax
from jax.experimental import pallas as pl
from jax.experimental.pallas import tpu as pltpu
from jax.experimental.pallas import tpu_sc as plsc

B, N = 2, 2048
D_IN, D = 128, 256
H, DH = 4, 64
GX, GY, GT = 128, 128, 200
TOT = GX * GY * GT
NJ = 32
BN = B * N

_offs = np.array(
    [[dx, dy, dt] for dx in (-1, 0, 1) for dy in (-1, 0, 1) for dt in (-1, 0, 1)],
    dtype=np.int32,
)
_OFFS_T = np.full((3, NJ), 512, np.int32)
_OFFS_T[:, :27] = _offs.T
_OFFS_F8 = np.zeros((NJ, 8), np.float32)
_OFFS_F8[:27, :3] = _offs.astype(np.float32)

PBLK = 256
CBLK = 64


def _ka_body(offs_t_ref, offs_f8_ref, feat_ref, coords_ref, Wf_ref, bf_ref,
             Wq_ref, bq_ref, Wp1_ref, bp1_ref, Wp2_ref, bp2_ref, Wk_ref,
             bk_ref, Wv_ref, bv_ref, fp_ref, q_ref, nlin_ref, kpe_ref,
             vpe_ref):
    feat = feat_ref[0]
    fp = jnp.dot(feat, Wf_ref[...], preferred_element_type=jnp.float32) + bf_ref[...]
    fp_ref[0] = fp
    q_ref[0] = jnp.dot(fp, Wq_ref[...], preferred_element_type=jnp.float32) + bq_ref[...]

    c = coords_ref[0]
    v0 = (jnp.clip(c[:, 0:1] / 256.0, 0.0, 1.0) * (GX - 1.0)).astype(jnp.int32)
    v1 = (jnp.clip(c[:, 1:2] / 256.0, 0.0, 1.0) * (GY - 1.0)).astype(jnp.int32)
    v2 = (jnp.clip(c[:, 2:3], 0.0, 1.0) * (GT - 1.0)).astype(jnp.int32)
    offs_t = offs_t_ref[...]
    nv0 = v0 + offs_t[0:1, :]
    nv1 = v1 + offs_t[1:2, :]
    nv2 = v2 + offs_t[2:3, :]
    inb = ((nv0 >= 0) & (nv0 < GX) & (nv1 >= 0) & (nv1 < GY)
           & (nv2 >= 0) & (nv2 < GT))
    nlin = nv0 * (GY * GT) + nv1 * GT + nv2
    nlin_ref[0] = jnp.where(inb, nlin, -1)

    @pl.when((pl.program_id(0) == 0) & (pl.program_id(1) == 0))
    def _():
        ph = jnp.dot(offs_f8_ref[...], Wp1_ref[...],
                     preferred_element_type=jnp.float32) + bp1_ref[...]
        pe = jnp.dot(jnp.maximum(ph, 0.0), Wp2_ref[...],
                     preferred_element_type=jnp.float32) + bp2_ref[...]
        kpe_ref[...] = jnp.dot(pe, Wk_ref[...],
                               preferred_element_type=jnp.float32) + bk_ref[...]
        vpe_ref[...] = jnp.dot(pe, Wv_ref[...],
                               preferred_element_type=jnp.float32) + bv_ref[...]


def _stage_a(feat, coords, Wf, bf, Wq, bq, Wp1_8, bp1, Wp2, bp2, Wk, bk, Wv,
             bv, interpret=False):
    full = lambda shp: pl.BlockSpec(shp, lambda b, i: (0,) * len(shp))
    grid = (B, N // PBLK)
    return pl.pallas_call(
        _ka_body,
        grid=grid,
        in_specs=[
            full((3, NJ)), full((NJ, 8)),
            pl.BlockSpec((1, PBLK, D_IN), lambda b, i: (b, i, 0)),
            pl.BlockSpec((1, PBLK, 3), lambda b, i: (b, i, 0)),
            full((D_IN, D)), full((1, D)), full((D, D)), full((1, D)),
            full((8, D // 2)), full((1, D // 2)), full((D // 2, D)),
            full((1, D)), full((D, D)), full((1, D)), full((D, D)),
            full((1, D)),
        ],
        out_specs=[
            pl.BlockSpec((1, PBLK, D), lambda b, i: (b, i, 0)),
            pl.BlockSpec((1, PBLK, D), lambda b, i: (b, i, 0)),
            pl.BlockSpec((1, PBLK, NJ), lambda b, i: (b, i, 0)),
            full((NJ, D)), full((NJ, D)),
        ],
        out_shape=[
            jax.ShapeDtypeStruct((B, N, D), jnp.float32),
            jax.ShapeDtypeStruct((B, N, D), jnp.float32),
            jax.ShapeDtypeStruct((B, N, NJ), jnp.int32),
            jax.ShapeDtypeStruct((NJ, D), jnp.float32),
            jax.ShapeDtypeStruct((NJ, D), jnp.float32),
        ],
        interpret=interpret,
    )(jnp.asarray(_OFFS_T), jnp.asarray(_OFFS_F8), feat, coords, Wf, bf, Wq,
      bq, Wp1_8, bp1, Wp2, bp2, Wk, bk, Wv, bv)


def _mesh():
    return plsc.VectorSubcoreMesh(core_axis_name="c", subcore_axis_name="s")


def _s1a_body(lin_hbm, tbl_hbm, idx_v, val_v, sem):
    c = lax.axis_index("c")
    s = lax.axis_index("s")
    wid = s * 2 + c
    gbase = wid * (BN // 32)
    b = gbase // N
    pltpu.sync_copy(lin_hbm.at[pl.ds(gbase, 128)], idx_v)

    def it(i, _):
        sl = pl.ds(i * 16, 16)
        idx_v[sl] = idx_v[sl] + b * TOT
        val_v[sl] = (gbase - b * N) + i * 16 + lax.iota(jnp.int32, 16)
        return 0

    lax.fori_loop(0, 8, it, 0)
    pltpu.async_copy(val_v, tbl_hbm.at[idx_v], sem).wait()


def _stage_s1a(lin_flat):
    return pl.kernel(
        _s1a_body,
        out_type=jax.ShapeDtypeStruct((B * TOT,), jnp.int32),
        mesh=_mesh(),
        scratch_types=[
            pltpu.VMEM((128,), jnp.int32),
            pltpu.VMEM((128,), jnp.int32),
            pltpu.SemaphoreType.DMA,
        ],
    )(lin_flat)


def _s1b_body(lin_hbm, tbl_hbm, rep_hbm, idx_v, rep_v, repf_v, sem):
    c = lax.axis_index("c")
    s = lax.axis_index("s")
    wid = s * 2 + c
    gbase = wid * (BN // 32)
    b = gbase // N
    pltpu.sync_copy(lin_hbm.at[pl.ds(gbase, 128)], idx_v)

    def it(i, _):
        sl = pl.ds(i * 16, 16)
        idx_v[sl] = idx_v[sl] + b * TOT
        return 0

    lax.fori_loop(0, 8, it, 0)
    pltpu.async_copy(tbl_hbm.at[idx_v], rep_v, sem).wait()

    def cv(i, _):
        sl = pl.ds(i * 16, 16)
        repf_v[sl] = rep_v[sl].astype(jnp.float32)
        return 0

    lax.fori_loop(0, 8, cv, 0)
    pltpu.sync_copy(repf_v, rep_hbm.at[pl.ds(gbase, 128)])


def _stage_s1b(lin_flat, tbl):
    return pl.kernel(
        _s1b_body,
        out_type=jax.ShapeDtypeStruct((BN,), jnp.float32),
        mesh=_mesh(),
        scratch_types=[
            pltpu.VMEM((128,), jnp.int32),
            pltpu.VMEM((128,), jnp.int32),
            pltpu.VMEM((128,), jnp.float32),
            pltpu.SemaphoreType.DMA,
        ],
    )(lin_flat, tbl)


def _kseg_body(rep_ref, fp_ref, sums_ref, cnt_ref):
    r = pl.program_id(1)
    rep = rep_ref[0]
    fp = fp_ref[0]
    riota = (lax.broadcasted_iota(jnp.int32, (1, PBLK), 1)
             + r * PBLK).astype(jnp.float32)
    acc = jnp.zeros((PBLK, D), jnp.float32)
    cacc = jnp.zeros((PBLK, 1), jnp.float32)
    ones = jnp.ones((PBLK, 1), jnp.float32)
    for nb in range(N // PBLK):
        selT = jnp.where(rep[nb * PBLK:(nb + 1) * PBLK] == riota, 1.0, 0.0)
        acc = acc + lax.dot_general(
            selT, fp[nb * PBLK:(nb + 1) * PBLK],
            (((0,), (0,)), ((), ())), preferred_element_type=jnp.float32)
        cacc = cacc + lax.dot_general(
            selT, ones, (((0,), (0,)), ((), ())),
            preferred_element_type=jnp.float32)
    sums_ref[0] = acc
    cnt_ref[0] = cacc


def _stage_seg(rep, fp, interpret=False):
    return pl.pallas_call(
        _kseg_body,
        grid=(B, N // PBLK),
        in_specs=[
            pl.BlockSpec((1, N, 1), lambda b, r: (b, 0, 0)),
            pl.BlockSpec((1, N, D), lambda b, r: (b, 0, 0)),
        ],
        out_specs=[
            pl.BlockSpec((1, PBLK, D), lambda b, r: (b, r, 0)),
            pl.BlockSpec((1, PBLK, 1), lambda b, r: (b, r, 0)),
        ],
        out_shape=[
            jax.ShapeDtypeStruct((B, N, D), jnp.float32),
            jax.ShapeDtypeStruct((B, N, 1), jnp.float32),
        ],
        interpret=interpret,
    )(rep, fp)


def _kb_body(sums_ref, cnt_ref, Wia_ref, bia_ref, Wk_ref, Wv_ref, kv_ref):
    cnt = cnt_ref[0]
    means = sums_ref[0] / jnp.maximum(cnt, 1.0)
    vf = jnp.maximum(
        jnp.dot(means, Wia_ref[...], preferred_element_type=jnp.float32)
        + bia_ref[...], 0.0)
    kvf = jnp.dot(vf, Wk_ref[...], preferred_element_type=jnp.float32)
    vvf = jnp.dot(vf, Wv_ref[...], preferred_element_type=jnp.float32)
    kb = lax.bitcast_convert_type(kvf.astype(jnp.bfloat16), jnp.uint16)
    vb = lax.bitcast_convert_type(vvf.astype(jnp.bfloat16), jnp.uint16)
    w = (kb.astype(jnp.uint32) << 16) | vb.astype(jnp.uint32)
    kv_ref[0] = lax.bitcast_convert_type(w, jnp.int32)


def _stage_b(sums, cnt, Wia, bia, Wk, Wv, interpret=False):
    full = lambda shp: pl.BlockSpec(shp, lambda b, i: (0,) * len(shp))
    return pl.pallas_call(
        _kb_body,
        grid=(B, N // PBLK),
        in_specs=[
            pl.BlockSpec((1, PBLK, D), lambda b, i: (b, i, 0)),
            pl.BlockSpec((1, PBLK, 1), lambda b, i: (b, i, 0)),
            full((D, D)), full((1, D)), full((D, D)), full((D, D)),
        ],
        out_specs=[
            pl.BlockSpec((1, PBLK, D), lambda b, i: (b, i, 0)),
        ],
        out_shape=[
            jax.ShapeDtypeStruct((B, N, D), jnp.int32),
        ],
        interpret=interpret,
    )(sums, cnt, Wia, bia, Wk, Wv)[0]


def _s2_body(nlin_hbm, lin_hbm, tbl_hbm, kv_hbm, kvrows_hbm, found_hbm,
             nl_v, gi_v, g_v, gc_v, lv_v, ri_v, fd_v, kv_v, sem):
    c = lax.axis_index("c")
    s = lax.axis_index("s")
    wid = s * 2 + c
    gbase = wid * (BN // 32)
    b = gbase // N

    def chunk(ch, _):
        rb = gbase * NJ + ch * 128
        pltpu.sync_copy(nlin_hbm.at[pl.ds(rb, 128)], nl_v)

        def lane(i, _):
            sl = pl.ds(i * 16, 16)
            gi_v[sl] = jnp.maximum(nl_v[sl], 0) + b * TOT
            return 0

        lax.fori_loop(0, 8, lane, 0)
        pltpu.async_copy(tbl_hbm.at[gi_v], g_v, sem).wait()

        def lane2(i, _):
            sl = pl.ds(i * 16, 16)
            gc_v[sl] = jnp.clip(g_v[sl], 0, N - 1) + b * N
            return 0

        lax.fori_loop(0, 8, lane2, 0)
        pltpu.async_copy(lin_hbm.at[gc_v], lv_v, sem).wait()

        def lane3(i, _):
            sl = pl.ds(i * 16, 16)
            nl = nl_v[sl]
            fnd = (nl >= 0) & (lv_v[sl] == nl)
            ri_v[sl] = jnp.where(fnd, gc_v[sl], b * N)
            fd_v[sl] = jnp.where(fnd, 1, 0)
            return 0

        lax.fori_loop(0, 8, lane3, 0)

        def fire(k, _):
            sl = pl.ds(k * 32, 32)
            pltpu.async_copy(kv_hbm.at[ri_v.at[sl]], kv_v.at[sl], sem)
            return 0

        lax.fori_loop(0, 4, fire, 0)

        def drain(k, _):
            sl = pl.ds(k * 32, 32)
            pltpu.make_async_copy(kv_hbm.at[ri_v.at[sl]], kv_v.at[sl],
                                  sem).wait()
            return 0

        lax.fori_loop(0, 4, drain, 0)
        pltpu.sync_copy(kv_v, kvrows_hbm.at[pl.ds(rb, 128)])
        pltpu.sync_copy(fd_v, found_hbm.at[pl.ds(rb, 128)])
        return 0

    lax.fori_loop(0, (BN // 32) * NJ // 128, chunk, 0)


def _stage_s2(nlin_flat, lin_flat, tbl, kv3):
    return pl.kernel(
        _s2_body,
        out_type=(
            jax.ShapeDtypeStruct((BN * NJ, 2, 128), jnp.int32),
            jax.ShapeDtypeStruct((BN * NJ,), jnp.int32),
        ),
        mesh=_mesh(),
        scratch_types=[
            pltpu.VMEM((128,), jnp.int32),
            pltpu.VMEM((128,), jnp.int32),
            pltpu.VMEM((128,), jnp.int32),
            pltpu.VMEM((128,), jnp.int32),
            pltpu.VMEM((128,), jnp.int32),
            pltpu.VMEM((128,), jnp.int32),
            pltpu.VMEM((128,), jnp.int32),
            pltpu.VMEM((128, 2, 128), jnp.int32),
            pltpu.SemaphoreType.DMA,
        ],
    )(nlin_flat, lin_flat, tbl, kv3)


def _kc_body(q_ref, fp_ref, kvrows_ref, found_ref, kpe_ref,
             vpe_ref, Wo_ref, bo_ref, gamma_ref, beta_ref, out_ref):
    R = CBLK * NJ
    f32 = jnp.float32
    r_i = lax.broadcasted_iota(jnp.int32, (R, 1), 0)
    grp = jnp.where(r_i // NJ == lax.broadcasted_iota(jnp.int32, (1, CBLK), 1),
                    1.0, 0.0)
    tile = jnp.where(r_i % NJ == lax.broadcasted_iota(jnp.int32, (1, NJ), 1),
                     1.0, 0.0)
    hm = jnp.where(
        lax.broadcasted_iota(jnp.int32, (D, 1), 0) // DH
        == lax.broadcasted_iota(jnp.int32, (1, H), 1), 1.0, 0.0)
    hm_t = jnp.where(
        lax.broadcasted_iota(jnp.int32, (H, 1), 0)
        == lax.broadcasted_iota(jnp.int32, (1, D), 1) // DH, 1.0, 0.0)
    mm = functools.partial(jnp.dot, preferred_element_type=f32)
    ct0 = lambda a, b: lax.dot_general(a, b, (((0,), (0,)), ((), ())),
                                       preferred_element_type=f32)
    wu = lax.bitcast_convert_type(kvrows_ref[0], jnp.uint32)
    kbf = lax.bitcast_convert_type((wu >> 16).astype(jnp.uint16), jnp.bfloat16)
    vbf = lax.bitcast_convert_type((wu & 0xFFFF).astype(jnp.uint16), jnp.bfloat16)
    krow = kbf.astype(f32) + mm(tile, kpe_ref[...])
    vrow = vbf.astype(f32) + mm(tile, vpe_ref[...])
    qe = mm(grp, q_ref[0])
    logits = mm(qe * krow, hm) * (1.0 / 8.0)
    fnd = found_ref[0]
    e = jnp.exp(jnp.where(fnd > 0, logits, -1e9))
    denom = mm(grp, ct0(grp, e))
    a_bc = mm(e / denom, hm_t)
    ctx = ct0(grp, a_bc * vrow)
    out = mm(ctx, Wo_ref[...]) + bo_ref[...]
    enh = fp_ref[0] + out
    mu = jnp.mean(enh, axis=-1, keepdims=True)
    var = jnp.mean((enh - mu) ** 2, axis=-1, keepdims=True)
    out_ref[0] = ((enh - mu) * lax.rsqrt(var + 1e-5) * gamma_ref[...]
                  + beta_ref[...])


def _stage_c(q, fp, kvrows, found, kpe, vpe, Wo, bo, gamma, beta,
             interpret=False):
    full = lambda shp: pl.BlockSpec(shp, lambda b, i: (0,) * len(shp))
    return pl.pallas_call(
        _kc_body,
        grid=(B, N // CBLK),
        in_specs=[
            pl.BlockSpec((1, CBLK, D), lambda b, i: (b, i, 0)),
            pl.BlockSpec((1, CBLK, D), lambda b, i: (b, i, 0)),
            pl.BlockSpec((1, CBLK * NJ, D), lambda b, i: (b, i, 0)),
            pl.BlockSpec((1, CBLK * NJ, 1), lambda b, i: (b, i, 0)),
            full((NJ, D)), full((NJ, D)), full((D, D)), full((1, D)),
            full((1, D)), full((1, D)),
        ],
        out_specs=[pl.BlockSpec((1, CBLK, D), lambda b, i: (b, i, 0))],
        out_shape=[jax.ShapeDtypeStruct((B, N, D), jnp.float32)],
        interpret=interpret,
    )(q, fp, kvrows, found, kpe, vpe, Wo, bo, gamma, beta)[0]


def kernel(features, coords, W_feat, b_feat, W_ia, b_ia, W_p1, b_p1, W_p2,
           b_p2, Wq, bq, Wk, bk, Wv, bv, Wo, bo, gamma, beta):
    b2 = lambda x: x.reshape(1, -1)
    Wp1_8 = jnp.zeros((8, D // 2), jnp.float32).at[:3, :].set(W_p1)

    fp, q, nlin, kpe, vpe = _stage_a(
        features, coords, W_feat, b2(b_feat), Wq, b2(bq), Wp1_8, b2(b_p1),
        W_p2, b2(b_p2), Wk, b2(bk), Wv, b2(bv))

    lin_flat = nlin[:, :, 13].reshape(BN)
    nlin_flat = nlin.reshape(BN * NJ)
    fp_flat = fp.reshape(BN, D)

    tbl = _stage_s1a(lin_flat)
    rep_f = _stage_s1b(lin_flat, tbl)
    sums, cnt = _stage_seg(rep_f.reshape(B, N, 1), fp)
    kv = _stage_b(sums, cnt, W_ia, b2(b_ia), Wk, Wv)
    kvrows, found = _stage_s2(nlin_flat, lin_flat, tbl,
                              kv.reshape(BN, 2, 128))
    return _stage_c(q, fp, kvrows.reshape(B, N * NJ, D),
                    found.reshape(B, N * NJ, 1),
                    kpe, vpe, Wo, b2(bo), b2(gamma), b2(beta))


# speedup vs baseline: 1.1743x; 1.1743x over previous
import functools

import numpy as np
import jax
import jax.numpy as jnp
from jax import lax
from jax.experimental import pallas as pl
from jax.experimental.pallas import tpu as pltpu
from jax.experimental.pallas import tpu_sc as plsc

B, N = 2, 2048
D_IN, D = 128, 256
H, DH = 4, 64
GX, GY, GT = 128, 128, 200
TOT = GX * GY * GT
NJ = 27
NJP = 32
BN = B * N

_offs = np.array(
    [[dx, dy, dt] for dx in (-1, 0, 1) for dy in (-1, 0, 1) for dt in (-1, 0, 1)],
    dtype=np.int32,
)
_OFFS_T = np.full((3, NJ), 512, np.int32)
_OFFS_T[:, :27] = _offs.T
_OFFS_F8 = np.zeros((NJP, 8), np.float32)
_OFFS_F8[:27, :3] = _offs.astype(np.float32)

PBLK = 256
CBLK = 64


def _ka_body(offs_t_ref, offs_f8_ref, feat_ref, coords_ref, Wf_ref, bf_ref,
             Wq_ref, bq_ref, Wp1_ref, bp1_ref, Wp2_ref, bp2_ref, Wk_ref,
             bk_ref, Wv_ref, bv_ref, fp_ref, q_ref, nlin_ref, kpe_ref,
             vpe_ref):
    feat = feat_ref[0]
    fp = jnp.dot(feat, Wf_ref[...], preferred_element_type=jnp.float32) + bf_ref[...]
    fp_ref[0] = fp
    q_ref[0] = jnp.dot(fp, Wq_ref[...], preferred_element_type=jnp.float32) + bq_ref[...]

    c = coords_ref[0]
    v0 = (jnp.clip(c[:, 0:1] / 256.0, 0.0, 1.0) * (GX - 1.0)).astype(jnp.int32)
    v1 = (jnp.clip(c[:, 1:2] / 256.0, 0.0, 1.0) * (GY - 1.0)).astype(jnp.int32)
    v2 = (jnp.clip(c[:, 2:3], 0.0, 1.0) * (GT - 1.0)).astype(jnp.int32)
    offs_t = offs_t_ref[...]
    nv0 = v0 + offs_t[0:1, :]
    nv1 = v1 + offs_t[1:2, :]
    nv2 = v2 + offs_t[2:3, :]
    inb = ((nv0 >= 0) & (nv0 < GX) & (nv1 >= 0) & (nv1 < GY)
           & (nv2 >= 0) & (nv2 < GT))
    nlin = nv0 * (GY * GT) + nv1 * GT + nv2
    nlin_ref[0] = jnp.where(inb, nlin, -1)

    @pl.when((pl.program_id(0) == 0) & (pl.program_id(1) == 0))
    def _():
        ph = jnp.dot(offs_f8_ref[...], Wp1_ref[...],
                     preferred_element_type=jnp.float32) + bp1_ref[...]
        pe = jnp.dot(jnp.maximum(ph, 0.0), Wp2_ref[...],
                     preferred_element_type=jnp.float32) + bp2_ref[...]
        kpe_ref[...] = jnp.dot(pe, Wk_ref[...],
                               preferred_element_type=jnp.float32) + bk_ref[...]
        vpe_ref[...] = jnp.dot(pe, Wv_ref[...],
                               preferred_element_type=jnp.float32) + bv_ref[...]


def _stage_a(feat, coords, Wf, bf, Wq, bq, Wp1_8, bp1, Wp2, bp2, Wk, bk, Wv,
             bv, interpret=False):
    full = lambda shp: pl.BlockSpec(shp, lambda b, i: (0,) * len(shp))
    grid = (B, N // PBLK)
    return pl.pallas_call(
        _ka_body,
        grid=grid,
        in_specs=[
            full((3, NJ)), full((NJP, 8)),
            pl.BlockSpec((1, PBLK, D_IN), lambda b, i: (b, i, 0)),
            pl.BlockSpec((1, PBLK, 3), lambda b, i: (b, i, 0)),
            full((D_IN, D)), full((1, D)), full((D, D)), full((1, D)),
            full((8, D // 2)), full((1, D // 2)), full((D // 2, D)),
            full((1, D)), full((D, D)), full((1, D)), full((D, D)),
            full((1, D)),
        ],
        out_specs=[
            pl.BlockSpec((1, PBLK, D), lambda b, i: (b, i, 0)),
            pl.BlockSpec((1, PBLK, D), lambda b, i: (b, i, 0)),
            pl.BlockSpec((1, PBLK, NJ), lambda b, i: (b, i, 0)),
            full((NJP, D)), full((NJP, D)),
        ],
        out_shape=[
            jax.ShapeDtypeStruct((B, N, D), jnp.float32),
            jax.ShapeDtypeStruct((B, N, D), jnp.float32),
            jax.ShapeDtypeStruct((B, N, NJ), jnp.int32),
            jax.ShapeDtypeStruct((NJP, D), jnp.float32),
            jax.ShapeDtypeStruct((NJP, D), jnp.float32),
        ],
        interpret=interpret,
    )(jnp.asarray(_OFFS_T), jnp.asarray(_OFFS_F8), feat, coords, Wf, bf, Wq,
      bq, Wp1_8, bp1, Wp2, bp2, Wk, bk, Wv, bv)


def _mesh():
    return plsc.VectorSubcoreMesh(core_axis_name="c", subcore_axis_name="s")


def _s1a_body(lin_hbm, tbl_hbm, idx_v, val_v, sem):
    c = lax.axis_index("c")
    s = lax.axis_index("s")
    wid = s * 2 + c
    gbase = wid * (BN // 32)
    b = gbase // N
    pltpu.sync_copy(lin_hbm.at[pl.ds(gbase, 128)], idx_v)

    def it(i, _):
        sl = pl.ds(i * 16, 16)
        idx_v[sl] = idx_v[sl] + b * TOT
        val_v[sl] = (gbase - b * N) + i * 16 + lax.iota(jnp.int32, 16)
        return 0

    lax.fori_loop(0, 8, it, 0)
    pltpu.async_copy(val_v, tbl_hbm.at[idx_v], sem).wait()


def _stage_s1a(lin_flat):
    return pl.kernel(
        _s1a_body,
        out_type=jax.ShapeDtypeStruct((B * TOT,), jnp.int32),
        mesh=_mesh(),
        scratch_types=[
            pltpu.VMEM((128,), jnp.int32),
            pltpu.VMEM((128,), jnp.int32),
            pltpu.SemaphoreType.DMA,
        ],
    )(lin_flat)


def _s1b_body(lin_hbm, tbl_hbm, rep_hbm, idx_v, rep_v, repf_v, sem):
    c = lax.axis_index("c")
    s = lax.axis_index("s")
    wid = s * 2 + c
    gbase = wid * (BN // 32)
    b = gbase // N
    pltpu.sync_copy(lin_hbm.at[pl.ds(gbase, 128)], idx_v)

    def it(i, _):
        sl = pl.ds(i * 16, 16)
        idx_v[sl] = idx_v[sl] + b * TOT
        return 0

    lax.fori_loop(0, 8, it, 0)
    pltpu.async_copy(tbl_hbm.at[idx_v], rep_v, sem).wait()

    def cv(i, _):
        sl = pl.ds(i * 16, 16)
        repf_v[sl] = rep_v[sl].astype(jnp.float32)
        return 0

    lax.fori_loop(0, 8, cv, 0)
    pltpu.sync_copy(repf_v, rep_hbm.at[pl.ds(gbase, 128)])


def _stage_s1b(lin_flat, tbl):
    return pl.kernel(
        _s1b_body,
        out_type=jax.ShapeDtypeStruct((BN,), jnp.float32),
        mesh=_mesh(),
        scratch_types=[
            pltpu.VMEM((128,), jnp.int32),
            pltpu.VMEM((128,), jnp.int32),
            pltpu.VMEM((128,), jnp.float32),
            pltpu.SemaphoreType.DMA,
        ],
    )(lin_flat, tbl)


def _kseg_body(rep_ref, fp_ref, sums_ref, cnt_ref):
    r = pl.program_id(1)
    rep = rep_ref[0]
    fp = fp_ref[0]
    riota = (lax.broadcasted_iota(jnp.int32, (1, PBLK), 1)
             + r * PBLK).astype(jnp.float32)
    acc = jnp.zeros((PBLK, D), jnp.float32)
    cacc = jnp.zeros((PBLK, 1), jnp.float32)
    ones = jnp.ones((PBLK, 1), jnp.float32)
    for nb in range(N // PBLK):
        selT = jnp.where(rep[nb * PBLK:(nb + 1) * PBLK] == riota, 1.0, 0.0)
        acc = acc + lax.dot_general(
            selT, fp[nb * PBLK:(nb + 1) * PBLK],
            (((0,), (0,)), ((), ())), preferred_element_type=jnp.float32)
        cacc = cacc + lax.dot_general(
            selT, ones, (((0,), (0,)), ((), ())),
            preferred_element_type=jnp.float32)
    sums_ref[0] = acc
    cnt_ref[0] = cacc


def _stage_seg(rep, fp, interpret=False):
    return pl.pallas_call(
        _kseg_body,
        grid=(B, N // PBLK),
        in_specs=[
            pl.BlockSpec((1, N, 1), lambda b, r: (b, 0, 0)),
            pl.BlockSpec((1, N, D), lambda b, r: (b, 0, 0)),
        ],
        out_specs=[
            pl.BlockSpec((1, PBLK, D), lambda b, r: (b, r, 0)),
            pl.BlockSpec((1, PBLK, 1), lambda b, r: (b, r, 0)),
        ],
        out_shape=[
            jax.ShapeDtypeStruct((B, N, D), jnp.float32),
            jax.ShapeDtypeStruct((B, N, 1), jnp.float32),
        ],
        interpret=interpret,
    )(rep, fp)


def _kb_body(sums_ref, cnt_ref, Wia_ref, bia_ref, Wk_ref, Wv_ref, kv_ref):
    cnt = cnt_ref[0]
    means = sums_ref[0] / jnp.maximum(cnt, 1.0)
    vf = jnp.maximum(
        jnp.dot(means, Wia_ref[...], preferred_element_type=jnp.float32)
        + bia_ref[...], 0.0)
    kvf = jnp.dot(vf, Wk_ref[...], preferred_element_type=jnp.float32)
    vvf = jnp.dot(vf, Wv_ref[...], preferred_element_type=jnp.float32)
    kb = lax.bitcast_convert_type(kvf.astype(jnp.bfloat16), jnp.uint16)
    vb = lax.bitcast_convert_type(vvf.astype(jnp.bfloat16), jnp.uint16)
    w = (kb.astype(jnp.uint32) << 16) | vb.astype(jnp.uint32)
    kv_ref[0] = lax.bitcast_convert_type(w, jnp.int32)


def _stage_b(sums, cnt, Wia, bia, Wk, Wv, interpret=False):
    full = lambda shp: pl.BlockSpec(shp, lambda b, i: (0,) * len(shp))
    return pl.pallas_call(
        _kb_body,
        grid=(B, N // PBLK),
        in_specs=[
            pl.BlockSpec((1, PBLK, D), lambda b, i: (b, i, 0)),
            pl.BlockSpec((1, PBLK, 1), lambda b, i: (b, i, 0)),
            full((D, D)), full((1, D)), full((D, D)), full((D, D)),
        ],
        out_specs=[
            pl.BlockSpec((1, PBLK, D), lambda b, i: (b, i, 0)),
        ],
        out_shape=[
            jax.ShapeDtypeStruct((B, N, D), jnp.int32),
        ],
        interpret=interpret,
    )(sums, cnt, Wia, bia, Wk, Wv)[0]


def _s2_body(nlin_hbm, lin_hbm, tbl_hbm, kv_hbm, kvrows_hbm, found_hbm,
             nl_v, gi_v, g_v, gc_v, lv_v, ri_v, fd_v, kv_v, sem):
    c = lax.axis_index("c")
    s = lax.axis_index("s")
    wid = s * 2 + c
    gbase = wid * (BN // 32)
    b = gbase // N

    def chunk(ch, _):
        rb = gbase * NJ + ch * 128
        pltpu.sync_copy(nlin_hbm.at[pl.ds(rb, 128)], nl_v)

        def lane(i, _):
            sl = pl.ds(i * 16, 16)
            gi_v[sl] = jnp.maximum(nl_v[sl], 0) + b * TOT
            return 0

        lax.fori_loop(0, 8, lane, 0)
        pltpu.async_copy(tbl_hbm.at[gi_v], g_v, sem).wait()

        def lane2(i, _):
            sl = pl.ds(i * 16, 16)
            gc_v[sl] = jnp.clip(g_v[sl], 0, N - 1) + b * N
            return 0

        lax.fori_loop(0, 8, lane2, 0)
        pltpu.async_copy(lin_hbm.at[gc_v], lv_v, sem).wait()

        def lane3(i, _):
            sl = pl.ds(i * 16, 16)
            nl = nl_v[sl]
            fnd = (nl >= 0) & (lv_v[sl] == nl)
            ri_v[sl] = jnp.where(fnd, gc_v[sl], b * N)
            fd_v[sl] = jnp.where(fnd, 1, 0)
            return 0

        lax.fori_loop(0, 8, lane3, 0)
        pltpu.async_copy(kv_hbm.at[ri_v], kv_v, sem).wait()
        pltpu.sync_copy(kv_v, kvrows_hbm.at[pl.ds(rb, 128)])
        pltpu.sync_copy(fd_v, found_hbm.at[pl.ds(rb, 128)])
        return 0

    lax.fori_loop(0, (BN // 32) * NJ // 128, chunk, 0)


def _stage_s2(nlin_flat, lin_flat, tbl, kv3):
    return pl.kernel(
        _s2_body,
        out_type=(
            jax.ShapeDtypeStruct((BN * NJ, 2, 128), jnp.int32),
            jax.ShapeDtypeStruct((BN * NJ,), jnp.int32),
        ),
        mesh=_mesh(),
        scratch_types=[
            pltpu.VMEM((128,), jnp.int32),
            pltpu.VMEM((128,), jnp.int32),
            pltpu.VMEM((128,), jnp.int32),
            pltpu.VMEM((128,), jnp.int32),
            pltpu.VMEM((128,), jnp.int32),
            pltpu.VMEM((128,), jnp.int32),
            pltpu.VMEM((128,), jnp.int32),
            pltpu.VMEM((128, 2, 128), jnp.int32),
            pltpu.SemaphoreType.DMA,
        ],
    )(nlin_flat, lin_flat, tbl, kv3)


def _kc_body(q_ref, fp_ref, kvrows_ref, found_ref, kpe_ref,
             vpe_ref, Wo_ref, bo_ref, gamma_ref, beta_ref, out_ref):
    R = CBLK * NJ
    f32 = jnp.float32
    r_i = lax.broadcasted_iota(jnp.int32, (R, 1), 0)
    grp = jnp.where(r_i // NJ == lax.broadcasted_iota(jnp.int32, (1, CBLK), 1),
                    1.0, 0.0)
    tile = jnp.where(r_i % NJ == lax.broadcasted_iota(jnp.int32, (1, NJP), 1),
                     1.0, 0.0)
    hm = jnp.where(
        lax.broadcasted_iota(jnp.int32, (D, 1), 0) // DH
        == lax.broadcasted_iota(jnp.int32, (1, H), 1), 1.0, 0.0)
    hm_t = jnp.where(
        lax.broadcasted_iota(jnp.int32, (H, 1), 0)
        == lax.broadcasted_iota(jnp.int32, (1, D), 1) // DH, 1.0, 0.0)
    mm = functools.partial(jnp.dot, preferred_element_type=f32)
    ct0 = lambda a, b: lax.dot_general(a, b, (((0,), (0,)), ((), ())),
                                       preferred_element_type=f32)
    wu = lax.bitcast_convert_type(kvrows_ref[0], jnp.uint32)
    kbf = lax.bitcast_convert_type((wu >> 16).astype(jnp.uint16), jnp.bfloat16)
    vbf = lax.bitcast_convert_type((wu & 0xFFFF).astype(jnp.uint16), jnp.bfloat16)
    krow = kbf.astype(f32) + mm(tile, kpe_ref[...])
    vrow = vbf.astype(f32) + mm(tile, vpe_ref[...])
    qe = mm(grp, q_ref[0])
    logits = mm(qe * krow, hm) * (1.0 / 8.0)
    fnd = found_ref[0]
    e = jnp.exp(jnp.where(fnd > 0, logits, -1e9))
    denom = mm(grp, ct0(grp, e))
    a_bc = mm(e / denom, hm_t)
    ctx = ct0(grp, a_bc * vrow)
    out = mm(ctx, Wo_ref[...]) + bo_ref[...]
    enh = fp_ref[0] + out
    mu = jnp.mean(enh, axis=-1, keepdims=True)
    var = jnp.mean((enh - mu) ** 2, axis=-1, keepdims=True)
    out_ref[0] = ((enh - mu) * lax.rsqrt(var + 1e-5) * gamma_ref[...]
                  + beta_ref[...])


def _stage_c(q, fp, kvrows, found, kpe, vpe, Wo, bo, gamma, beta,
             interpret=False):
    full = lambda shp: pl.BlockSpec(shp, lambda b, i: (0,) * len(shp))
    return pl.pallas_call(
        _kc_body,
        grid=(B, N // CBLK),
        in_specs=[
            pl.BlockSpec((1, CBLK, D), lambda b, i: (b, i, 0)),
            pl.BlockSpec((1, CBLK, D), lambda b, i: (b, i, 0)),
            pl.BlockSpec((1, CBLK * NJ, D), lambda b, i: (b, i, 0)),
            pl.BlockSpec((1, CBLK * NJ, 1), lambda b, i: (b, i, 0)),
            full((NJP, D)), full((NJP, D)), full((D, D)), full((1, D)),
            full((1, D)), full((1, D)),
        ],
        out_specs=[pl.BlockSpec((1, CBLK, D), lambda b, i: (b, i, 0))],
        out_shape=[jax.ShapeDtypeStruct((B, N, D), jnp.float32)],
        interpret=interpret,
    )(q, fp, kvrows, found, kpe, vpe, Wo, bo, gamma, beta)[0]


def kernel(features, coords, W_feat, b_feat, W_ia, b_ia, W_p1, b_p1, W_p2,
           b_p2, Wq, bq, Wk, bk, Wv, bv, Wo, bo, gamma, beta):
    b2 = lambda x: x.reshape(1, -1)
    Wp1_8 = jnp.zeros((8, D // 2), jnp.float32).at[:3, :].set(W_p1)

    fp, q, nlin, kpe, vpe = _stage_a(
        features, coords, W_feat, b2(b_feat), Wq, b2(bq), Wp1_8, b2(b_p1),
        W_p2, b2(b_p2), Wk, b2(bk), Wv, b2(bv))

    lin_flat = nlin[:, :, 13].reshape(BN)
    nlin_flat = nlin.reshape(BN * NJ)
    fp_flat = fp.reshape(BN, D)

    tbl = _stage_s1a(lin_flat)
    rep_f = _stage_s1b(lin_flat, tbl)
    sums, cnt = _stage_seg(rep_f.reshape(B, N, 1), fp)
    kv = _stage_b(sums, cnt, W_ia, b2(b_ia), Wk, Wv)
    kvrows, found = _stage_s2(nlin_flat, lin_flat, tbl,
                              kv.reshape(BN, 2, 128))
    return _stage_c(q, fp, kvrows.reshape(B, N * NJ, D),
                    found.reshape(B, N * NJ, 1),
                    kpe, vpe, Wo, b2(bo), b2(gamma), b2(beta))
